# Initial kernel scaffold; baseline (speedup 1.0000x reference)
import jax
import jax.numpy as jnp
from jax.experimental import pallas as pl


def kernel(idx, targets, table):
    raise NotImplementedError("write your pallas kernel here")


# speedup vs baseline: 1.4255x; 1.4255x over previous
import functools

import jax
import jax.numpy as jnp
from jax import lax
from jax.experimental import pallas as pl
from jax.experimental.pallas import tpu as pltpu
from jax.experimental.pallas import tpu_sc as plsc

VOCAB = 1000
B, T = 4096, 50
BT = B * T

NC, NS, L = 2, 16, 16
NW = NC * NS
RPW = BT // NW
R = 64
CP = 1024


def _lse_body(table_ref, out_ref):
    t = table_ref[...]
    m = jnp.max(t, axis=1, keepdims=True)
    out_ref[...] = m + jnp.log(jnp.sum(jnp.exp(t - m), axis=1, keepdims=True))


def _row_lse(table):
    return pl.pallas_call(
        _lse_body,
        out_shape=jax.ShapeDtypeStruct((VOCAB, 1), jnp.float32),
    )(table)


_MESH = plsc.VectorSubcoreMesh(core_axis_name="c", subcore_axis_name="s")


@functools.partial(
    pl.kernel,
    out_type=(
        jax.ShapeDtypeStruct((BT, VOCAB), jnp.float32),
        jax.ShapeDtypeStruct((NW, L), jnp.float32),
    ),
    mesh=_MESH,
    compiler_params=pltpu.CompilerParams(
        needs_layout_passes=False, use_tc_tiling_on_sc=False),
    scratch_types=[
        pltpu.VMEM((RPW,), jnp.int32),
        pltpu.VMEM((RPW,), jnp.int32),
        pltpu.VMEM((1, VOCAB), jnp.float32),
        pltpu.VMEM((R, VOCAB), jnp.float32),
        pltpu.VMEM((L,), jnp.float32),
        pltpu.SemaphoreType.DMA,
    ],
)
def _sc_gather_loss(table_hbm, idx_hbm, tgt_hbm, lse_hbm,
                    logits_hbm, part_hbm,
                    idx_v, tgt_v, lse_v, rows_v, acc_v, sem):
    wid = lax.axis_index("s") * NC + lax.axis_index("c")
    base = wid * RPW
    pltpu.sync_copy(idx_hbm.at[pl.ds(base, RPW)], idx_v)
    pltpu.sync_copy(tgt_hbm.at[pl.ds(base, RPW)], tgt_v)
    pltpu.sync_copy(lse_hbm, lse_v)

    lane = lax.iota(jnp.int32, L)
    zero = jnp.zeros((L,), jnp.int32)

    def chunk_body(g, acc):
        off = g * R
        pltpu.async_copy(table_hbm.at[idx_v.at[pl.ds(off, R)]], rows_v, sem).wait()
        pltpu.sync_copy(rows_v, logits_hbm.at[pl.ds(base + off, R)])
        for j in range(R // L):
            iv = idx_v[pl.ds(off + j * L, L)]
            tv = tgt_v[pl.ds(off + j * L, L)]
            lsev = plsc.load_gather(lse_v, [zero, iv])
            picks = plsc.load_gather(rows_v, [lane + j * L, tv])
            acc = acc + (lsev - picks)
        return acc

    acc = lax.fori_loop(0, RPW // R, chunk_body, jnp.zeros((L,), jnp.float32))
    acc_v[...] = acc
    pltpu.sync_copy(acc_v, part_hbm.at[wid])


def kernel(idx, targets, table):
    lse = _row_lse(table).reshape(1, VOCAB)
    idx_flat = idx.reshape(BT)
    tgt_flat = targets.reshape(BT)
    logits_flat, parts = _sc_gather_loss(table, idx_flat, tgt_flat, lse)
    loss = jnp.sum(parts) / BT
    return (logits_flat.reshape(B, T, VOCAB), loss)
